# baseline (speedup 1.0000x reference)
import functools

import jax
import jax.numpy as jnp
from jax import lax
from jax.experimental import pallas as pl
from jax.experimental.pallas import tpu as pltpu
from jax.experimental.pallas import tpu_sc as plsc

N = 8192
D = 1024
F = 4096
E = 8
K = 2

BM = 256
BF = 1024
NF = F // BF
NB = 72
M_PAD = NB * BM

BR = 1024
NEG = -1e30


def _router_body(x_ref, wg_ref, idx_ref, w_ref):
    xb = x_ref[...]
    g = jnp.dot(xb, wg_ref[...], preferred_element_type=jnp.float32)
    lane = lax.broadcasted_iota(jnp.int32, g.shape, 1)
    valid = lane < E
    gm = jnp.where(valid, g, NEG)
    m1 = jnp.max(gm, axis=1, keepdims=True)
    i1 = jnp.min(jnp.where(gm == m1, lane, 999), axis=1, keepdims=True)
    g2 = jnp.where(lane == i1, NEG, gm)
    m2 = jnp.max(g2, axis=1, keepdims=True)
    i2 = jnp.min(jnp.where(g2 == m2, lane, 999), axis=1, keepdims=True)
    e2 = jnp.exp(m2 - m1)
    w1 = 1.0 / (1.0 + e2)
    w2 = e2 * w1
    lane8 = lax.broadcasted_iota(jnp.int32, (BR, E), 1)
    idx_ref[...] = jnp.where(lane8 == 0, i1, i2)
    w_ref[...] = jnp.where(lane8 == 0, w1, w2)


def _router(x_flat, Wg):
    wg_pad = jnp.zeros((D, 128), jnp.float32).at[:, :E].set(Wg)
    return pl.pallas_call(
        _router_body,
        grid=(N // BR,),
        in_specs=[
            pl.BlockSpec((BR, D), lambda i: (i, 0)),
            pl.BlockSpec((D, 128), lambda i: (0, 0)),
        ],
        out_specs=[
            pl.BlockSpec((BR, E), lambda i: (i, 0)),
            pl.BlockSpec((BR, E), lambda i: (i, 0)),
        ],
        out_shape=[
            jax.ShapeDtypeStruct((N, E), jnp.int32),
            jax.ShapeDtypeStruct((N, E), jnp.float32),
        ],
    )(x_flat, wg_pad)


def _metadata(top_idx, top_w):
    e_flat = top_idx.T.reshape(-1)
    w_flat = top_w.T.reshape(-1)
    order = jnp.argsort(e_flat, stable=True)
    e_sorted = e_flat[order]
    offs = jnp.searchsorted(e_sorted, jnp.arange(E, dtype=e_sorted.dtype),
                            side="left").astype(jnp.int32)
    counts = jnp.diff(jnp.concatenate([offs, jnp.array([N * K], jnp.int32)]))
    nb_e = (counts + BM - 1) // BM
    blk_end = jnp.cumsum(nb_e).astype(jnp.int32)
    blk_start = blk_end - nb_e
    row_start = blk_start * BM
    s = jnp.arange(N * K, dtype=jnp.int32)
    pos_s = row_start[e_sorted] + (s - offs[e_sorted])
    gather_idx = jnp.zeros((M_PAD,), jnp.int32).at[pos_s].set(
        (order % N).astype(jnp.int32))
    w_pad = jnp.zeros((M_PAD,), jnp.float32).at[pos_s].set(w_flat[order])
    inv_pos = jnp.zeros((N * K,), jnp.int32).at[order].set(pos_s)
    block_expert = jnp.minimum(
        jnp.searchsorted(blk_end, jnp.arange(NB, dtype=jnp.int32),
                         side="right").astype(jnp.int32), E - 1)
    return gather_idx, w_pad, inv_pos, block_expert


def _ffn_body(be_ref, xs_ref, w1_ref, w3_ref, w2_ref, wrow_ref, out_ref):
    j = pl.program_id(1)
    xb = xs_ref[...].astype(jnp.bfloat16)
    a = jnp.dot(xb, w1_ref[0], preferred_element_type=jnp.float32)
    c = jnp.dot(xb, w3_ref[0], preferred_element_type=jnp.float32)
    h = (a * jax.nn.sigmoid(a) * c).astype(jnp.bfloat16)
    part = jnp.dot(h, w2_ref[0], preferred_element_type=jnp.float32)

    @pl.when(j == 0)
    def _():
        out_ref[...] = jnp.zeros_like(out_ref)

    out_ref[...] += part

    @pl.when(j == NF - 1)
    def _():
        out_ref[...] *= wrow_ref[:, 0:1]


def _grouped_ffn(xs, W1b, W3b, W2b, w2d, block_expert):
    grid_spec = pltpu.PrefetchScalarGridSpec(
        num_scalar_prefetch=1,
        grid=(NB, NF),
        in_specs=[
            pl.BlockSpec((BM, D), lambda i, j, be: (i, 0)),
            pl.BlockSpec((1, D, BF), lambda i, j, be: (be[i], 0, j)),
            pl.BlockSpec((1, D, BF), lambda i, j, be: (be[i], 0, j)),
            pl.BlockSpec((1, BF, D), lambda i, j, be: (be[i], j, 0)),
            pl.BlockSpec((BM, 128), lambda i, j, be: (i, 0)),
        ],
        out_specs=pl.BlockSpec((BM, D), lambda i, j, be: (i, 0)),
    )
    return pl.pallas_call(
        _ffn_body,
        grid_spec=grid_spec,
        out_shape=jax.ShapeDtypeStruct((M_PAD, D), jnp.float32),
        compiler_params=pltpu.CompilerParams(
            dimension_semantics=("arbitrary", "arbitrary")),
    )(block_expert, xs, W1b, W3b, W2b, w2d)


NC = 2
NS = 16
NW = NC * NS

GCH = 64
CCH = 32


def _sc_gather(table, gather_idx, n_rows):
    rpw = n_rows // NW
    gch = GCH
    while rpw % gch:
        gch //= 2
    nch = rpw // gch
    mesh = plsc.VectorSubcoreMesh(core_axis_name="c", subcore_axis_name="s")

    @functools.partial(
        pl.kernel, mesh=mesh,
        out_type=jax.ShapeDtypeStruct((n_rows, D), jnp.float32),
        scratch_types=[
            pltpu.VMEM((gch,), jnp.int32),
            pltpu.VMEM((gch, D), jnp.float32),
            pltpu.SemaphoreType.DMA,
        ],
    )
    def k(x_hbm, idx_hbm, out_hbm, idx_v, rows_v, sem):
        wid = lax.axis_index("s") * NC + lax.axis_index("c")
        base = wid * rpw

        def body(i, _):
            off = base + i * gch
            pltpu.sync_copy(idx_hbm.at[pl.ds(off, gch)], idx_v)
            pltpu.async_copy(x_hbm.at[idx_v], rows_v, sem).wait()
            pltpu.sync_copy(rows_v, out_hbm.at[pl.ds(off, gch)])
            return 0

        lax.fori_loop(0, nch, body, 0)

    return k(table, gather_idx)


BR2 = 512


def _pair_add_body(g0_ref, g1_ref, o_ref):
    o_ref[...] = g0_ref[...] + g1_ref[...]


def _pair_add(gcat):
    nb = N // BR2
    return pl.pallas_call(
        _pair_add_body,
        grid=(nb,),
        in_specs=[
            pl.BlockSpec((BR2, D), lambda i: (i, 0)),
            pl.BlockSpec((BR2, D), lambda i: (i + N // BR2, 0)),
        ],
        out_specs=pl.BlockSpec((BR2, D), lambda i: (i, 0)),
        out_shape=jax.ShapeDtypeStruct((N, D), jnp.float32),
    )(gcat, gcat)


def kernel(x, Wg, W1, W2, W3):
    Bb, Tt, Dd = x.shape
    x_flat = x.reshape(-1, Dd)
    top_idx, top_w = _router(x_flat, Wg)
    gather_idx, w_pad, inv_pos, block_expert = _metadata(
        top_idx[:, :K], top_w[:, :K])

    W1b = W1.astype(jnp.bfloat16)
    W3b = W3.astype(jnp.bfloat16)
    W2b = W2.astype(jnp.bfloat16)

    xs = _sc_gather(x_flat, gather_idx, M_PAD)
    w2d = jnp.broadcast_to(w_pad[:, None], (M_PAD, 128))

    ys = _grouped_ffn(xs, W1b, W3b, W2b, w2d, block_expert)

    gcat = _sc_gather(ys, inv_pos, N * K)
    out = _pair_add(gcat)
    return out.reshape(Bb, Tt, Dd)


# speedup vs baseline: 1.0402x; 1.0402x over previous
import functools

import jax
import jax.numpy as jnp
from jax import lax
from jax.experimental import pallas as pl
from jax.experimental.pallas import tpu as pltpu
from jax.experimental.pallas import tpu_sc as plsc

N = 8192
D = 1024
F = 4096
E = 8
K = 2

BM = 512
BF = 1024
NF = F // BF
NB = 39
M_PAD = NB * BM

BR = 1024
NEG = -1e30


def _router_body(x_ref, wg_ref, idx_ref, w_ref):
    xb = x_ref[...]
    g = jnp.dot(xb, wg_ref[...], preferred_element_type=jnp.float32)
    lane = lax.broadcasted_iota(jnp.int32, g.shape, 1)
    valid = lane < E
    gm = jnp.where(valid, g, NEG)
    m1 = jnp.max(gm, axis=1, keepdims=True)
    i1 = jnp.min(jnp.where(gm == m1, lane, 999), axis=1, keepdims=True)
    g2 = jnp.where(lane == i1, NEG, gm)
    m2 = jnp.max(g2, axis=1, keepdims=True)
    i2 = jnp.min(jnp.where(g2 == m2, lane, 999), axis=1, keepdims=True)
    e2 = jnp.exp(m2 - m1)
    w1 = 1.0 / (1.0 + e2)
    w2 = e2 * w1
    lane8 = lax.broadcasted_iota(jnp.int32, (BR, E), 1)
    idx_ref[...] = jnp.where(lane8 == 0, i1, i2)
    w_ref[...] = jnp.where(lane8 == 0, w1, w2)


def _router(x_flat, Wg):
    wg_pad = jnp.zeros((D, 128), jnp.float32).at[:, :E].set(Wg)
    return pl.pallas_call(
        _router_body,
        grid=(N // BR,),
        in_specs=[
            pl.BlockSpec((BR, D), lambda i: (i, 0)),
            pl.BlockSpec((D, 128), lambda i: (0, 0)),
        ],
        out_specs=[
            pl.BlockSpec((BR, E), lambda i: (i, 0)),
            pl.BlockSpec((BR, E), lambda i: (i, 0)),
        ],
        out_shape=[
            jax.ShapeDtypeStruct((N, E), jnp.int32),
            jax.ShapeDtypeStruct((N, E), jnp.float32),
        ],
    )(x_flat, wg_pad)


def _metadata(top_idx, top_w):
    e_flat = top_idx.T.reshape(-1)
    w_flat = top_w.T.reshape(-1)
    order = jnp.argsort(e_flat, stable=True)
    e_sorted = e_flat[order]
    offs = jnp.searchsorted(e_sorted, jnp.arange(E, dtype=e_sorted.dtype),
                            side="left").astype(jnp.int32)
    counts = jnp.diff(jnp.concatenate([offs, jnp.array([N * K], jnp.int32)]))
    nb_e = (counts + BM - 1) // BM
    blk_end = jnp.cumsum(nb_e).astype(jnp.int32)
    blk_start = blk_end - nb_e
    row_start = blk_start * BM
    s = jnp.arange(N * K, dtype=jnp.int32)
    pos_s = row_start[e_sorted] + (s - offs[e_sorted])
    gather_idx = jnp.zeros((M_PAD,), jnp.int32).at[pos_s].set(
        (order % N).astype(jnp.int32))
    w_pad = jnp.zeros((M_PAD,), jnp.float32).at[pos_s].set(w_flat[order])
    inv_pos = jnp.zeros((N * K,), jnp.int32).at[order].set(pos_s)
    block_expert = jnp.minimum(
        jnp.searchsorted(blk_end, jnp.arange(NB, dtype=jnp.int32),
                         side="right").astype(jnp.int32), E - 1)
    return gather_idx, w_pad, inv_pos, block_expert


def _ffn_body(be_ref, xs_ref, w1_ref, w3_ref, w2_ref, wrow_ref, out_ref):
    j = pl.program_id(1)
    xb = xs_ref[...].astype(jnp.bfloat16)
    a = jnp.dot(xb, w1_ref[0], preferred_element_type=jnp.float32)
    c = jnp.dot(xb, w3_ref[0], preferred_element_type=jnp.float32)
    h = (a * jax.nn.sigmoid(a) * c).astype(jnp.bfloat16)
    part = jnp.dot(h, w2_ref[0], preferred_element_type=jnp.float32)

    @pl.when(j == 0)
    def _():
        out_ref[...] = jnp.zeros_like(out_ref)

    out_ref[...] += part

    @pl.when(j == NF - 1)
    def _():
        out_ref[...] *= wrow_ref[:, 0:1]


def _grouped_ffn(xs, W1b, W3b, W2b, w2d, block_expert):
    grid_spec = pltpu.PrefetchScalarGridSpec(
        num_scalar_prefetch=1,
        grid=(NB, NF),
        in_specs=[
            pl.BlockSpec((BM, D), lambda i, j, be: (i, 0)),
            pl.BlockSpec((1, D, BF), lambda i, j, be: (be[i], 0, j)),
            pl.BlockSpec((1, D, BF), lambda i, j, be: (be[i], 0, j)),
            pl.BlockSpec((1, BF, D), lambda i, j, be: (be[i], j, 0)),
            pl.BlockSpec((BM, 128), lambda i, j, be: (i, 0)),
        ],
        out_specs=pl.BlockSpec((BM, D), lambda i, j, be: (i, 0)),
    )
    return pl.pallas_call(
        _ffn_body,
        grid_spec=grid_spec,
        out_shape=jax.ShapeDtypeStruct((M_PAD, D), jnp.float32),
        compiler_params=pltpu.CompilerParams(
            dimension_semantics=("arbitrary", "arbitrary")),
    )(block_expert, xs, W1b, W3b, W2b, w2d)


NC = 2
NS = 16
NW = NC * NS

GCH = 64
CCH = 32


def _sc_gather(table, gather_idx, n_rows):
    rpw = n_rows // NW
    gch = 8
    for c in range(104, 7, -8):
        if rpw % c == 0:
            gch = c
            break
    nch = rpw // gch
    mesh = plsc.VectorSubcoreMesh(core_axis_name="c", subcore_axis_name="s")

    @functools.partial(
        pl.kernel, mesh=mesh,
        out_type=jax.ShapeDtypeStruct((n_rows, D), jnp.float32),
        scratch_types=[
            pltpu.VMEM((gch,), jnp.int32),
            pltpu.VMEM((gch, D), jnp.float32),
            pltpu.SemaphoreType.DMA,
        ],
    )
    def k(x_hbm, idx_hbm, out_hbm, idx_v, rows_v, sem):
        wid = lax.axis_index("s") * NC + lax.axis_index("c")
        base = wid * rpw

        def body(i, _):
            off = base + i * gch
            pltpu.sync_copy(idx_hbm.at[pl.ds(off, gch)], idx_v)
            pltpu.async_copy(x_hbm.at[idx_v], rows_v, sem).wait()
            pltpu.sync_copy(rows_v, out_hbm.at[pl.ds(off, gch)])
            return 0

        lax.fori_loop(0, nch, body, 0)

    return k(table, gather_idx)


BR2 = 512


def _pair_add_body(g0_ref, g1_ref, o_ref):
    o_ref[...] = g0_ref[...] + g1_ref[...]


def _pair_add(gcat):
    nb = N // BR2
    return pl.pallas_call(
        _pair_add_body,
        grid=(nb,),
        in_specs=[
            pl.BlockSpec((BR2, D), lambda i: (i, 0)),
            pl.BlockSpec((BR2, D), lambda i: (i + N // BR2, 0)),
        ],
        out_specs=pl.BlockSpec((BR2, D), lambda i: (i, 0)),
        out_shape=jax.ShapeDtypeStruct((N, D), jnp.float32),
    )(gcat, gcat)


def kernel(x, Wg, W1, W2, W3):
    Bb, Tt, Dd = x.shape
    x_flat = x.reshape(-1, Dd)
    top_idx, top_w = _router(x_flat, Wg)
    gather_idx, w_pad, inv_pos, block_expert = _metadata(
        top_idx[:, :K], top_w[:, :K])

    W1b = W1.astype(jnp.bfloat16)
    W3b = W3.astype(jnp.bfloat16)
    W2b = W2.astype(jnp.bfloat16)

    xs = _sc_gather(x_flat, gather_idx, M_PAD)
    w2d = jnp.broadcast_to(w_pad[:, None], (M_PAD, 128))

    ys = _grouped_ffn(xs, W1b, W3b, W2b, w2d, block_expert)

    gcat = _sc_gather(ys, inv_pos, N * K)
    out = _pair_add(gcat)
    return out.reshape(Bb, Tt, Dd)


# speedup vs baseline: 1.0723x; 1.0309x over previous
import functools

import jax
import jax.numpy as jnp
from jax import lax
from jax.experimental import pallas as pl
from jax.experimental.pallas import tpu as pltpu
from jax.experimental.pallas import tpu_sc as plsc

N = 8192
D = 1024
F = 4096
E = 8
K = 2

BM = 512
BF = 1024
NF = F // BF
NB = 39
M_PAD = NB * BM

BR = 1024
NEG = -1e30


def _router_body(x_ref, wg_ref, idx_ref, w_ref, xp_ref):
    xb = x_ref[...]
    lo = lax.bitcast_convert_type(
        xb[:, :D // 2].astype(jnp.bfloat16), jnp.uint16).astype(jnp.uint32)
    hi = lax.bitcast_convert_type(
        xb[:, D // 2:].astype(jnp.bfloat16), jnp.uint16).astype(jnp.uint32)
    xp_ref[...] = lax.bitcast_convert_type(lo | (hi << 16), jnp.int32)
    g = jnp.dot(xb, wg_ref[...], preferred_element_type=jnp.float32)
    lane = lax.broadcasted_iota(jnp.int32, g.shape, 1)
    valid = lane < E
    gm = jnp.where(valid, g, NEG)
    m1 = jnp.max(gm, axis=1, keepdims=True)
    i1 = jnp.min(jnp.where(gm == m1, lane, 999), axis=1, keepdims=True)
    g2 = jnp.where(lane == i1, NEG, gm)
    m2 = jnp.max(g2, axis=1, keepdims=True)
    i2 = jnp.min(jnp.where(g2 == m2, lane, 999), axis=1, keepdims=True)
    e2 = jnp.exp(m2 - m1)
    w1 = 1.0 / (1.0 + e2)
    w2 = e2 * w1
    lane8 = lax.broadcasted_iota(jnp.int32, (BR, E), 1)
    idx_ref[...] = jnp.where(lane8 == 0, i1, i2)
    w_ref[...] = jnp.where(lane8 == 0, w1, w2)


def _router(x_flat, Wg):
    wg_pad = jnp.zeros((D, 128), jnp.float32).at[:, :E].set(Wg)
    return pl.pallas_call(
        _router_body,
        grid=(N // BR,),
        in_specs=[
            pl.BlockSpec((BR, D), lambda i: (i, 0)),
            pl.BlockSpec((D, 128), lambda i: (0, 0)),
        ],
        out_specs=[
            pl.BlockSpec((BR, E), lambda i: (i, 0)),
            pl.BlockSpec((BR, E), lambda i: (i, 0)),
            pl.BlockSpec((BR, D // 2), lambda i: (i, 0)),
        ],
        out_shape=[
            jax.ShapeDtypeStruct((N, E), jnp.int32),
            jax.ShapeDtypeStruct((N, E), jnp.float32),
            jax.ShapeDtypeStruct((N, D // 2), jnp.int32),
        ],
    )(x_flat, wg_pad)


def _metadata(top_idx, top_w):
    e_flat = top_idx.T.reshape(-1)
    w_flat = top_w.T.reshape(-1)
    order = jnp.argsort(e_flat, stable=True)
    e_sorted = e_flat[order]
    offs = jnp.searchsorted(e_sorted, jnp.arange(E, dtype=e_sorted.dtype),
                            side="left").astype(jnp.int32)
    counts = jnp.diff(jnp.concatenate([offs, jnp.array([N * K], jnp.int32)]))
    nb_e = (counts + BM - 1) // BM
    blk_end = jnp.cumsum(nb_e).astype(jnp.int32)
    blk_start = blk_end - nb_e
    row_start = blk_start * BM
    s = jnp.arange(N * K, dtype=jnp.int32)
    pos_s = row_start[e_sorted] + (s - offs[e_sorted])
    gather_idx = jnp.zeros((M_PAD,), jnp.int32).at[pos_s].set(
        (order % N).astype(jnp.int32))
    w_pad = jnp.zeros((M_PAD,), jnp.float32).at[pos_s].set(w_flat[order])
    inv_pos = jnp.zeros((N * K,), jnp.int32).at[order].set(pos_s)
    block_expert = jnp.minimum(
        jnp.searchsorted(blk_end, jnp.arange(NB, dtype=jnp.int32),
                         side="right").astype(jnp.int32), E - 1)
    return gather_idx, w_pad, inv_pos, block_expert


def _ffn_body(be_ref, xs_ref, w1_ref, w3_ref, w2_ref, wrow_ref, out_ref):
    j = pl.program_id(1)
    xu = lax.bitcast_convert_type(xs_ref[...], jnp.uint32)
    lo = lax.bitcast_convert_type(
        (xu & 0xFFFF).astype(jnp.uint16), jnp.bfloat16)
    hi = lax.bitcast_convert_type(
        (xu >> 16).astype(jnp.uint16), jnp.bfloat16)
    xb = jnp.concatenate([lo, hi], axis=1)
    a = jnp.dot(xb, w1_ref[0], preferred_element_type=jnp.float32)
    c = jnp.dot(xb, w3_ref[0], preferred_element_type=jnp.float32)
    h = (a * jax.nn.sigmoid(a) * c).astype(jnp.bfloat16)
    part = jnp.dot(h, w2_ref[0], preferred_element_type=jnp.float32)

    @pl.when(j == 0)
    def _():
        out_ref[...] = jnp.zeros_like(out_ref)

    out_ref[...] += part

    @pl.when(j == NF - 1)
    def _():
        out_ref[...] *= wrow_ref[:, 0:1]


def _grouped_ffn(xs, W1b, W3b, W2b, w2d, block_expert):
    grid_spec = pltpu.PrefetchScalarGridSpec(
        num_scalar_prefetch=1,
        grid=(NB, NF),
        in_specs=[
            pl.BlockSpec((BM, D // 2), lambda i, j, be: (i, 0)),
            pl.BlockSpec((1, D, BF), lambda i, j, be: (be[i], 0, j)),
            pl.BlockSpec((1, D, BF), lambda i, j, be: (be[i], 0, j)),
            pl.BlockSpec((1, BF, D), lambda i, j, be: (be[i], j, 0)),
            pl.BlockSpec((BM, 128), lambda i, j, be: (i, 0)),
        ],
        out_specs=pl.BlockSpec((BM, D), lambda i, j, be: (i, 0)),
    )
    return pl.pallas_call(
        _ffn_body,
        grid_spec=grid_spec,
        out_shape=jax.ShapeDtypeStruct((M_PAD, D), jnp.float32),
        compiler_params=pltpu.CompilerParams(
            dimension_semantics=("arbitrary", "arbitrary")),
    )(block_expert, xs, W1b, W3b, W2b, w2d)


NC = 2
NS = 16
NW = NC * NS

GCH = 64
CCH = 32


def _sc_gather(table, gather_idx, n_rows, row_elems, dtype):
    rpw = n_rows // NW
    row_bytes = row_elems * 4
    max_c = min(104, (500 * 1024) // (2 * row_bytes))
    gch = 8
    for c in range(max_c - max_c % 8, 7, -8):
        if rpw % c == 0 and (rpw // c) % 2 == 0:
            gch = c
            break
    nch = rpw // gch
    mesh = plsc.VectorSubcoreMesh(core_axis_name="c", subcore_axis_name="s")

    @functools.partial(
        pl.kernel, mesh=mesh,
        out_type=jax.ShapeDtypeStruct((n_rows, row_elems), dtype),
        scratch_types=[
            pltpu.VMEM((gch,), jnp.int32),
            pltpu.VMEM((gch,), jnp.int32),
            pltpu.VMEM((gch, row_elems), dtype),
            pltpu.VMEM((gch, row_elems), dtype),
            pltpu.SemaphoreType.DMA,
            pltpu.SemaphoreType.DMA,
        ],
    )
    def k(x_hbm, idx_hbm, out_hbm, ia, ib, ra, rb, sa, sb):
        wid = lax.axis_index("s") * NC + lax.axis_index("c")
        base = wid * rpw

        def fire(i, idx_v, rows_v, sem):
            off = base + i * gch
            pltpu.sync_copy(idx_hbm.at[pl.ds(off, gch)], idx_v)
            pltpu.async_copy(x_hbm.at[idx_v], rows_v, sem)

        def drain(i, rows_v, sem):
            off = base + i * gch
            pltpu.make_async_copy(x_hbm.at[pl.ds(0, gch)], rows_v, sem).wait()
            pltpu.sync_copy(rows_v, out_hbm.at[pl.ds(off, gch)])

        fire(0, ia, ra, sa)

        def body(h, _):
            i = h * 2
            fire(i + 1, ib, rb, sb)
            drain(i, ra, sa)

            @pl.when(i + 2 < nch)
            def _():
                fire(i + 2, ia, ra, sa)

            drain(i + 1, rb, sb)
            return 0

        lax.fori_loop(0, nch // 2, body, 0)

    return k(table, gather_idx)


BR2 = 512


def _pair_add_body(g0_ref, g1_ref, o_ref):
    o_ref[...] = g0_ref[...] + g1_ref[...]


def _pair_add(gcat):
    nb = N // BR2
    return pl.pallas_call(
        _pair_add_body,
        grid=(nb,),
        in_specs=[
            pl.BlockSpec((BR2, D), lambda i: (i, 0)),
            pl.BlockSpec((BR2, D), lambda i: (i + N // BR2, 0)),
        ],
        out_specs=pl.BlockSpec((BR2, D), lambda i: (i, 0)),
        out_shape=jax.ShapeDtypeStruct((N, D), jnp.float32),
    )(gcat, gcat)


def kernel(x, Wg, W1, W2, W3):
    Bb, Tt, Dd = x.shape
    x_flat = x.reshape(-1, Dd)
    top_idx, top_w, xpack = _router(x_flat, Wg)
    gather_idx, w_pad, inv_pos, block_expert = _metadata(
        top_idx[:, :K], top_w[:, :K])

    W1b = W1.astype(jnp.bfloat16)
    W3b = W3.astype(jnp.bfloat16)
    W2b = W2.astype(jnp.bfloat16)

    xs = _sc_gather(xpack, gather_idx, M_PAD, D // 2, jnp.int32)
    w2d = jnp.broadcast_to(w_pad[:, None], (M_PAD, 128))

    ys = _grouped_ffn(xs, W1b, W3b, W2b, w2d, block_expert)

    gcat = _sc_gather(ys, inv_pos, N * K, D, jnp.float32)
    out = _pair_add(gcat)
    return out.reshape(Bb, Tt, Dd)


# speedup vs baseline: 1.0886x; 1.0152x over previous
import functools

import jax
import jax.numpy as jnp
from jax import lax
from jax.experimental import pallas as pl
from jax.experimental.pallas import tpu as pltpu
from jax.experimental.pallas import tpu_sc as plsc

N = 8192
D = 1024
F = 4096
E = 8
K = 2

BM = 512
BF = 1024
NF = F // BF
NB = 39
M_PAD = NB * BM

BR = 1024
NEG = -1e30


def _router_body(x_ref, wg_ref, idx_ref, w_ref, xp_ref):
    xb = x_ref[...]
    lo = lax.bitcast_convert_type(
        xb[:, :D // 2].astype(jnp.bfloat16), jnp.uint16).astype(jnp.uint32)
    hi = lax.bitcast_convert_type(
        xb[:, D // 2:].astype(jnp.bfloat16), jnp.uint16).astype(jnp.uint32)
    xp_ref[...] = lax.bitcast_convert_type(lo | (hi << 16), jnp.int32)
    g = jnp.dot(xb, wg_ref[...], preferred_element_type=jnp.float32)
    lane = lax.broadcasted_iota(jnp.int32, g.shape, 1)
    valid = lane < E
    gm = jnp.where(valid, g, NEG)
    m1 = jnp.max(gm, axis=1, keepdims=True)
    i1 = jnp.min(jnp.where(gm == m1, lane, 999), axis=1, keepdims=True)
    g2 = jnp.where(lane == i1, NEG, gm)
    m2 = jnp.max(g2, axis=1, keepdims=True)
    i2 = jnp.min(jnp.where(g2 == m2, lane, 999), axis=1, keepdims=True)
    e2 = jnp.exp(m2 - m1)
    w1 = 1.0 / (1.0 + e2)
    w2 = e2 * w1
    lane8 = lax.broadcasted_iota(jnp.int32, (BR, E), 1)
    idx_ref[...] = jnp.where(lane8 == 0, i1, i2)
    w_ref[...] = jnp.where(lane8 == 0, w1, w2)


def _router(x_flat, Wg):
    wg_pad = jnp.zeros((D, 128), jnp.float32).at[:, :E].set(Wg)
    return pl.pallas_call(
        _router_body,
        grid=(N // BR,),
        in_specs=[
            pl.BlockSpec((BR, D), lambda i: (i, 0)),
            pl.BlockSpec((D, 128), lambda i: (0, 0)),
        ],
        out_specs=[
            pl.BlockSpec((BR, E), lambda i: (i, 0)),
            pl.BlockSpec((BR, E), lambda i: (i, 0)),
            pl.BlockSpec((BR, D // 2), lambda i: (i, 0)),
        ],
        out_shape=[
            jax.ShapeDtypeStruct((N, E), jnp.int32),
            jax.ShapeDtypeStruct((N, E), jnp.float32),
            jax.ShapeDtypeStruct((N, D // 2), jnp.int32),
        ],
    )(x_flat, wg_pad)


def _metadata(top_idx, top_w):
    e_flat = top_idx.T.reshape(-1)
    w_flat = top_w.T.reshape(-1)
    order = jnp.argsort(e_flat, stable=True)
    e_sorted = e_flat[order]
    offs = jnp.searchsorted(e_sorted, jnp.arange(E, dtype=e_sorted.dtype),
                            side="left").astype(jnp.int32)
    counts = jnp.diff(jnp.concatenate([offs, jnp.array([N * K], jnp.int32)]))
    nb_e = (counts + BM - 1) // BM
    blk_end = jnp.cumsum(nb_e).astype(jnp.int32)
    blk_start = blk_end - nb_e
    row_start = blk_start * BM
    s = jnp.arange(N * K, dtype=jnp.int32)
    pos_s = row_start[e_sorted] + (s - offs[e_sorted])
    gather_idx = jnp.zeros((M_PAD,), jnp.int32).at[pos_s].set(
        (order % N).astype(jnp.int32))
    w_pad = jnp.zeros((M_PAD,), jnp.float32).at[pos_s].set(w_flat[order])
    inv_pos = jnp.zeros((N * K,), jnp.int32).at[order].set(pos_s)
    block_expert = jnp.minimum(
        jnp.searchsorted(blk_end, jnp.arange(NB, dtype=jnp.int32),
                         side="right").astype(jnp.int32), E - 1)
    return gather_idx, w_pad, inv_pos, block_expert


def _pack16(y):
    lo = lax.bitcast_convert_type(
        y[:, :D // 2].astype(jnp.bfloat16), jnp.uint16).astype(jnp.uint32)
    hi = lax.bitcast_convert_type(
        y[:, D // 2:].astype(jnp.bfloat16), jnp.uint16).astype(jnp.uint32)
    return lax.bitcast_convert_type(lo | (hi << 16), jnp.int32)


def _unpack16(xi):
    xu = lax.bitcast_convert_type(xi, jnp.uint32)
    lo = lax.bitcast_convert_type(
        (xu & 0xFFFF).astype(jnp.uint16), jnp.bfloat16)
    hi = lax.bitcast_convert_type(
        (xu >> 16).astype(jnp.uint16), jnp.bfloat16)
    return jnp.concatenate([lo, hi], axis=1)


def _ffn_body(be_ref, xs_ref, w1_ref, w3_ref, w2_ref, wrow_ref, out_ref,
              acc_ref):
    j = pl.program_id(1)
    xb = _unpack16(xs_ref[...])
    a = jnp.dot(xb, w1_ref[0], preferred_element_type=jnp.float32)
    c = jnp.dot(xb, w3_ref[0], preferred_element_type=jnp.float32)
    h = (a * jax.nn.sigmoid(a) * c).astype(jnp.bfloat16)
    part = jnp.dot(h, w2_ref[0], preferred_element_type=jnp.float32)

    @pl.when(j == 0)
    def _():
        acc_ref[...] = jnp.zeros_like(acc_ref)

    acc_ref[...] += part

    @pl.when(j == NF - 1)
    def _():
        out_ref[...] = _pack16(acc_ref[...] * wrow_ref[:, 0:1])


def _grouped_ffn(xs, W1b, W3b, W2b, w2d, block_expert):
    grid_spec = pltpu.PrefetchScalarGridSpec(
        num_scalar_prefetch=1,
        grid=(NB, NF),
        in_specs=[
            pl.BlockSpec((BM, D // 2), lambda i, j, be: (i, 0)),
            pl.BlockSpec((1, D, BF), lambda i, j, be: (be[i], 0, j)),
            pl.BlockSpec((1, D, BF), lambda i, j, be: (be[i], 0, j)),
            pl.BlockSpec((1, BF, D), lambda i, j, be: (be[i], j, 0)),
            pl.BlockSpec((BM, 128), lambda i, j, be: (i, 0)),
        ],
        out_specs=pl.BlockSpec((BM, D // 2), lambda i, j, be: (i, 0)),
        scratch_shapes=[pltpu.VMEM((BM, D), jnp.float32)],
    )
    return pl.pallas_call(
        _ffn_body,
        grid_spec=grid_spec,
        out_shape=jax.ShapeDtypeStruct((M_PAD, D // 2), jnp.int32),
        compiler_params=pltpu.CompilerParams(
            dimension_semantics=("arbitrary", "arbitrary")),
    )(block_expert, xs, W1b, W3b, W2b, w2d)


NC = 2
NS = 16
NW = NC * NS

GCH = 64
CCH = 32


def _sc_gather(table, gather_idx, n_rows, row_elems, dtype):
    rpw = n_rows // NW
    row_bytes = row_elems * 4
    max_c = min(104, (500 * 1024) // (2 * row_bytes))
    gch = 8
    for c in range(max_c - max_c % 8, 7, -8):
        if rpw % c == 0 and (rpw // c) % 2 == 0:
            gch = c
            break
    nch = rpw // gch
    mesh = plsc.VectorSubcoreMesh(core_axis_name="c", subcore_axis_name="s")

    @functools.partial(
        pl.kernel, mesh=mesh,
        out_type=jax.ShapeDtypeStruct((n_rows, row_elems), dtype),
        scratch_types=[
            pltpu.VMEM((rpw,), jnp.int32),
            pltpu.VMEM((gch, row_elems), dtype),
            pltpu.VMEM((gch, row_elems), dtype),
            pltpu.SemaphoreType.DMA,
            pltpu.SemaphoreType.DMA,
        ],
    )
    def k(x_hbm, idx_hbm, out_hbm, idx_all, ra, rb, sa, sb):
        wid = lax.axis_index("s") * NC + lax.axis_index("c")
        base = wid * rpw
        pltpu.sync_copy(idx_hbm.at[pl.ds(base, rpw)], idx_all)

        def fire(i, rows_v, sem):
            pltpu.async_copy(
                x_hbm.at[idx_all.at[pl.ds(i * gch, gch)]], rows_v, sem)

        def drain(i, rows_v, sem):
            off = base + i * gch
            pltpu.make_async_copy(x_hbm.at[pl.ds(0, gch)], rows_v, sem).wait()
            pltpu.sync_copy(rows_v, out_hbm.at[pl.ds(off, gch)])

        fire(0, ra, sa)

        def body(h, _):
            i = h * 2
            fire(i + 1, rb, sb)
            drain(i, ra, sa)

            @pl.when(i + 2 < nch)
            def _():
                fire(i + 2, ra, sa)

            drain(i + 1, rb, sb)
            return 0

        lax.fori_loop(0, nch // 2, body, 0)

    return k(table, gather_idx)


BR2 = 512


def _pair_add_body(g0_ref, g1_ref, o_ref):
    y0 = _unpack16(g0_ref[...]).astype(jnp.float32)
    y1 = _unpack16(g1_ref[...]).astype(jnp.float32)
    o_ref[...] = y0 + y1


def _pair_add(gcat):
    nb = N // BR2
    return pl.pallas_call(
        _pair_add_body,
        grid=(nb,),
        in_specs=[
            pl.BlockSpec((BR2, D // 2), lambda i: (i, 0)),
            pl.BlockSpec((BR2, D // 2), lambda i: (i + N // BR2, 0)),
        ],
        out_specs=pl.BlockSpec((BR2, D), lambda i: (i, 0)),
        out_shape=jax.ShapeDtypeStruct((N, D), jnp.float32),
    )(gcat, gcat)


def kernel(x, Wg, W1, W2, W3):
    Bb, Tt, Dd = x.shape
    x_flat = x.reshape(-1, Dd)
    top_idx, top_w, xpack = _router(x_flat, Wg)
    gather_idx, w_pad, inv_pos, block_expert = _metadata(
        top_idx[:, :K], top_w[:, :K])

    W1b = W1.astype(jnp.bfloat16)
    W3b = W3.astype(jnp.bfloat16)
    W2b = W2.astype(jnp.bfloat16)

    xs = _sc_gather(xpack, gather_idx, M_PAD, D // 2, jnp.int32)
    w2d = jnp.broadcast_to(w_pad[:, None], (M_PAD, 128))

    ys = _grouped_ffn(xs, W1b, W3b, W2b, w2d, block_expert)

    gcat = _sc_gather(ys, inv_pos, N * K, D // 2, jnp.int32)
    out = _pair_add(gcat)
    return out.reshape(Bb, Tt, Dd)


# speedup vs baseline: 1.1801x; 1.0841x over previous
import functools

import jax
import jax.numpy as jnp
from jax import lax
from jax.experimental import pallas as pl
from jax.experimental.pallas import tpu as pltpu
from jax.experimental.pallas import tpu_sc as plsc

N = 8192
D = 1024
F = 4096
E = 8
K = 2

BM = 512
BF = 1024
NF = F // BF
NB = 39
M_PAD = NB * BM

BR = 1024
NEG = -1e30


def _router_body(x_ref, wg_ref, idx_ref, w_ref, xp_ref):
    xb = x_ref[...]
    lo = lax.bitcast_convert_type(
        xb[:, :D // 2].astype(jnp.bfloat16), jnp.uint16).astype(jnp.uint32)
    hi = lax.bitcast_convert_type(
        xb[:, D // 2:].astype(jnp.bfloat16), jnp.uint16).astype(jnp.uint32)
    xp_ref[...] = lax.bitcast_convert_type(lo | (hi << 16), jnp.int32)
    g = jnp.dot(xb, wg_ref[...], preferred_element_type=jnp.float32)
    lane = lax.broadcasted_iota(jnp.int32, g.shape, 1)
    valid = lane < E
    gm = jnp.where(valid, g, NEG)
    m1 = jnp.max(gm, axis=1, keepdims=True)
    i1 = jnp.min(jnp.where(gm == m1, lane, 999), axis=1, keepdims=True)
    g2 = jnp.where(lane == i1, NEG, gm)
    m2 = jnp.max(g2, axis=1, keepdims=True)
    i2 = jnp.min(jnp.where(g2 == m2, lane, 999), axis=1, keepdims=True)
    e2 = jnp.exp(m2 - m1)
    w1 = 1.0 / (1.0 + e2)
    w2 = e2 * w1
    lane8 = lax.broadcasted_iota(jnp.int32, (BR, E), 1)
    idx_ref[...] = jnp.where(lane8 == 0, i1, i2)
    w_ref[...] = jnp.where(lane8 == 0, w1, w2)


def _router(x_flat, Wg):
    wg_pad = jnp.zeros((D, 128), jnp.float32).at[:, :E].set(Wg)
    return pl.pallas_call(
        _router_body,
        grid=(N // BR,),
        in_specs=[
            pl.BlockSpec((BR, D), lambda i: (i, 0)),
            pl.BlockSpec((D, 128), lambda i: (0, 0)),
        ],
        out_specs=[
            pl.BlockSpec((BR, E), lambda i: (i, 0)),
            pl.BlockSpec((BR, E), lambda i: (i, 0)),
            pl.BlockSpec((BR, D // 2), lambda i: (i, 0)),
        ],
        out_shape=[
            jax.ShapeDtypeStruct((N, E), jnp.int32),
            jax.ShapeDtypeStruct((N, E), jnp.float32),
            jax.ShapeDtypeStruct((N, D // 2), jnp.int32),
        ],
    )(x_flat, wg_pad)


def _metadata(top_idx, top_w):
    e_flat = top_idx.T.reshape(-1)
    w_flat = top_w.T.reshape(-1)
    order = jnp.argsort(e_flat, stable=True)
    e_sorted = e_flat[order]
    offs = jnp.searchsorted(e_sorted, jnp.arange(E, dtype=e_sorted.dtype),
                            side="left").astype(jnp.int32)
    counts = jnp.diff(jnp.concatenate([offs, jnp.array([N * K], jnp.int32)]))
    nb_e = (counts + BM - 1) // BM
    blk_end = jnp.cumsum(nb_e).astype(jnp.int32)
    blk_start = blk_end - nb_e
    row_start = blk_start * BM
    s = jnp.arange(N * K, dtype=jnp.int32)
    pos_s = row_start[e_sorted] + (s - offs[e_sorted])
    inv_order = jnp.argsort(order).astype(jnp.int32)
    inv_pos = pos_s[inv_order]
    p = jnp.arange(M_PAD, dtype=jnp.int32)
    be_row = jnp.minimum(
        jnp.searchsorted(blk_end, p // BM, side="right").astype(jnp.int32),
        E - 1)
    in_grp = p - row_start[be_row]
    valid = in_grp < counts[be_row]
    rank = jnp.clip(offs[be_row] + in_grp, 0, N * K - 1)
    src = order[rank].astype(jnp.int32)
    gather_idx = jnp.where(valid, src % N, 0).astype(jnp.int32)
    w_pad = jnp.where(valid, w_flat[src], 0.0)
    block_expert = be_row[::BM]
    return gather_idx, w_pad, inv_pos, block_expert


def _pack16(y):
    lo = lax.bitcast_convert_type(
        y[:, :D // 2].astype(jnp.bfloat16), jnp.uint16).astype(jnp.uint32)
    hi = lax.bitcast_convert_type(
        y[:, D // 2:].astype(jnp.bfloat16), jnp.uint16).astype(jnp.uint32)
    return lax.bitcast_convert_type(lo | (hi << 16), jnp.int32)


def _unpack16(xi):
    xu = lax.bitcast_convert_type(xi, jnp.uint32)
    lo = lax.bitcast_convert_type(
        (xu & 0xFFFF).astype(jnp.uint16), jnp.bfloat16)
    hi = lax.bitcast_convert_type(
        (xu >> 16).astype(jnp.uint16), jnp.bfloat16)
    return jnp.concatenate([lo, hi], axis=1)


def _ffn_body(be_ref, xs_ref, w1_ref, w3_ref, w2_ref, wrow_ref, out_ref,
              acc_ref):
    j = pl.program_id(1)
    xb = _unpack16(xs_ref[...])
    a = jnp.dot(xb, w1_ref[0], preferred_element_type=jnp.float32)
    c = jnp.dot(xb, w3_ref[0], preferred_element_type=jnp.float32)
    h = (a * jax.nn.sigmoid(a) * c).astype(jnp.bfloat16)
    part = jnp.dot(h, w2_ref[0], preferred_element_type=jnp.float32)

    @pl.when(j == 0)
    def _():
        acc_ref[...] = jnp.zeros_like(acc_ref)

    acc_ref[...] += part

    @pl.when(j == NF - 1)
    def _():
        out_ref[...] = _pack16(acc_ref[...] * wrow_ref[:, 0:1])


def _grouped_ffn(xs, W1b, W3b, W2b, w2d, block_expert):
    grid_spec = pltpu.PrefetchScalarGridSpec(
        num_scalar_prefetch=1,
        grid=(NB, NF),
        in_specs=[
            pl.BlockSpec((BM, D // 2), lambda i, j, be: (i, 0)),
            pl.BlockSpec((1, D, BF), lambda i, j, be: (be[i], 0, j)),
            pl.BlockSpec((1, D, BF), lambda i, j, be: (be[i], 0, j)),
            pl.BlockSpec((1, BF, D), lambda i, j, be: (be[i], j, 0)),
            pl.BlockSpec((BM, 128), lambda i, j, be: (i, 0)),
        ],
        out_specs=pl.BlockSpec((BM, D // 2), lambda i, j, be: (i, 0)),
        scratch_shapes=[pltpu.VMEM((BM, D), jnp.float32)],
    )
    return pl.pallas_call(
        _ffn_body,
        grid_spec=grid_spec,
        out_shape=jax.ShapeDtypeStruct((M_PAD, D // 2), jnp.int32),
        compiler_params=pltpu.CompilerParams(
            dimension_semantics=("arbitrary", "arbitrary")),
    )(block_expert, xs, W1b, W3b, W2b, w2d)


NC = 2
NS = 16
NW = NC * NS

GCH = 64
CCH = 32


def _sc_gather(table, gather_idx, n_rows, row_elems, dtype):
    rpw = n_rows // NW
    row_bytes = row_elems * 4
    max_c = min(104, (500 * 1024) // (2 * row_bytes))
    gch = 8
    for c in range(max_c - max_c % 8, 7, -8):
        if rpw % c == 0 and (rpw // c) % 2 == 0:
            gch = c
            break
    nch = rpw // gch
    mesh = plsc.VectorSubcoreMesh(core_axis_name="c", subcore_axis_name="s")

    @functools.partial(
        pl.kernel, mesh=mesh,
        out_type=jax.ShapeDtypeStruct((n_rows, row_elems), dtype),
        scratch_types=[
            pltpu.VMEM((rpw,), jnp.int32),
            pltpu.VMEM((gch, row_elems), dtype),
            pltpu.VMEM((gch, row_elems), dtype),
            pltpu.SemaphoreType.DMA,
            pltpu.SemaphoreType.DMA,
        ],
    )
    def k(x_hbm, idx_hbm, out_hbm, idx_all, ra, rb, sa, sb):
        wid = lax.axis_index("s") * NC + lax.axis_index("c")
        base = wid * rpw
        pltpu.sync_copy(idx_hbm.at[pl.ds(base, rpw)], idx_all)

        def fire(i, rows_v, sem):
            pltpu.async_copy(
                x_hbm.at[idx_all.at[pl.ds(i * gch, gch)]], rows_v, sem)

        def drain(i, rows_v, sem):
            off = base + i * gch
            pltpu.make_async_copy(x_hbm.at[pl.ds(0, gch)], rows_v, sem).wait()
            pltpu.sync_copy(rows_v, out_hbm.at[pl.ds(off, gch)])

        fire(0, ra, sa)

        def body(h, _):
            i = h * 2
            fire(i + 1, rb, sb)
            drain(i, ra, sa)

            @pl.when(i + 2 < nch)
            def _():
                fire(i + 2, ra, sa)

            drain(i + 1, rb, sb)
            return 0

        lax.fori_loop(0, nch // 2, body, 0)

    return k(table, gather_idx)


BR2 = 512


def _pair_add_body(g0_ref, g1_ref, o_ref):
    y0 = _unpack16(g0_ref[...]).astype(jnp.float32)
    y1 = _unpack16(g1_ref[...]).astype(jnp.float32)
    o_ref[...] = y0 + y1


def _pair_add(gcat):
    nb = N // BR2
    return pl.pallas_call(
        _pair_add_body,
        grid=(nb,),
        in_specs=[
            pl.BlockSpec((BR2, D // 2), lambda i: (i, 0)),
            pl.BlockSpec((BR2, D // 2), lambda i: (i + N // BR2, 0)),
        ],
        out_specs=pl.BlockSpec((BR2, D), lambda i: (i, 0)),
        out_shape=jax.ShapeDtypeStruct((N, D), jnp.float32),
    )(gcat, gcat)


def kernel(x, Wg, W1, W2, W3):
    Bb, Tt, Dd = x.shape
    x_flat = x.reshape(-1, Dd)
    top_idx, top_w, xpack = _router(x_flat, Wg)
    gather_idx, w_pad, inv_pos, block_expert = _metadata(
        top_idx[:, :K], top_w[:, :K])

    W1b = W1.astype(jnp.bfloat16)
    W3b = W3.astype(jnp.bfloat16)
    W2b = W2.astype(jnp.bfloat16)

    xs = _sc_gather(xpack, gather_idx, M_PAD, D // 2, jnp.int32)
    w2d = jnp.broadcast_to(w_pad[:, None], (M_PAD, 128))

    ys = _grouped_ffn(xs, W1b, W3b, W2b, w2d, block_expert)

    gcat = _sc_gather(ys, inv_pos, N * K, D // 2, jnp.int32)
    out = _pair_add(gcat)
    return out.reshape(Bb, Tt, Dd)


# speedup vs baseline: 1.2923x; 1.0951x over previous
import functools

import jax
import jax.numpy as jnp
from jax import lax
from jax.experimental import pallas as pl
from jax.experimental.pallas import tpu as pltpu
from jax.experimental.pallas import tpu_sc as plsc

N = 8192
D = 1024
F = 4096
E = 8
K = 2

BM = 512
BF = 1024
NF = F // BF
NB = 39
M_PAD = NB * BM

BR = 1024
NEG = -1e30


def _router_body(x_ref, wg_ref, idx_ref, w_ref, xp_ref):
    xb = x_ref[...]
    lo = lax.bitcast_convert_type(
        xb[:, :D // 2].astype(jnp.bfloat16), jnp.uint16).astype(jnp.uint32)
    hi = lax.bitcast_convert_type(
        xb[:, D // 2:].astype(jnp.bfloat16), jnp.uint16).astype(jnp.uint32)
    xp_ref[...] = lax.bitcast_convert_type(lo | (hi << 16), jnp.int32)
    g = jnp.dot(xb, wg_ref[...], preferred_element_type=jnp.float32)
    lane = lax.broadcasted_iota(jnp.int32, g.shape, 1)
    valid = lane < E
    gm = jnp.where(valid, g, NEG)
    m1 = jnp.max(gm, axis=1, keepdims=True)
    i1 = jnp.min(jnp.where(gm == m1, lane, 999), axis=1, keepdims=True)
    g2 = jnp.where(lane == i1, NEG, gm)
    m2 = jnp.max(g2, axis=1, keepdims=True)
    i2 = jnp.min(jnp.where(g2 == m2, lane, 999), axis=1, keepdims=True)
    e2 = jnp.exp(m2 - m1)
    w1 = 1.0 / (1.0 + e2)
    w2 = e2 * w1
    lane8 = lax.broadcasted_iota(jnp.int32, (BR, E), 1)
    idx_ref[...] = jnp.where(lane8 == 0, i1, i2)
    w_ref[...] = jnp.where(lane8 == 0, w1, w2)


def _router(x_flat, Wg):
    wg_pad = jnp.zeros((D, 128), jnp.float32).at[:, :E].set(Wg)
    return pl.pallas_call(
        _router_body,
        grid=(N // BR,),
        in_specs=[
            pl.BlockSpec((BR, D), lambda i: (i, 0)),
            pl.BlockSpec((D, 128), lambda i: (0, 0)),
        ],
        out_specs=[
            pl.BlockSpec((BR, E), lambda i: (i, 0)),
            pl.BlockSpec((BR, E), lambda i: (i, 0)),
            pl.BlockSpec((BR, D // 2), lambda i: (i, 0)),
        ],
        out_shape=[
            jax.ShapeDtypeStruct((N, E), jnp.int32),
            jax.ShapeDtypeStruct((N, E), jnp.float32),
            jax.ShapeDtypeStruct((N, D // 2), jnp.int32),
        ],
    )(x_flat, wg_pad)


def _metadata(top_idx, top_w):
    e_flat = top_idx.T.reshape(-1)
    w_flat = top_w.T.reshape(-1)
    order = jnp.argsort(e_flat, stable=True)
    e_sorted = e_flat[order]
    offs = jnp.searchsorted(e_sorted, jnp.arange(E, dtype=e_sorted.dtype),
                            side="left").astype(jnp.int32)
    counts = jnp.diff(jnp.concatenate([offs, jnp.array([N * K], jnp.int32)]))
    nb_e = (counts + BM - 1) // BM
    blk_end = jnp.cumsum(nb_e).astype(jnp.int32)
    blk_start = blk_end - nb_e
    row_start = blk_start * BM
    s = jnp.arange(N * K, dtype=jnp.int32)
    pos_s = row_start[e_sorted] + (s - offs[e_sorted])
    onehot = (e_flat[:, None] == jnp.arange(E, dtype=e_flat.dtype)[None, :])
    csum = jnp.cumsum(onehot.astype(jnp.int32), axis=0)
    within = jnp.take_along_axis(
        csum, e_flat[:, None].astype(jnp.int32), axis=1)[:, 0] - 1
    inv_pos = row_start[e_flat] + within
    p = jnp.arange(M_PAD, dtype=jnp.int32)
    be_row = jnp.minimum(
        jnp.searchsorted(blk_end, p // BM, side="right").astype(jnp.int32),
        E - 1)
    in_grp = p - row_start[be_row]
    valid = in_grp < counts[be_row]
    rank = jnp.clip(offs[be_row] + in_grp, 0, N * K - 1)
    src = order[rank].astype(jnp.int32)
    gather_idx = jnp.where(valid, src % N, 0).astype(jnp.int32)
    w_pad = jnp.where(valid, w_flat[src], 0.0)
    block_expert = be_row[::BM]
    return gather_idx, w_pad, inv_pos, block_expert


def _pack16(y):
    lo = lax.bitcast_convert_type(
        y[:, :D // 2].astype(jnp.bfloat16), jnp.uint16).astype(jnp.uint32)
    hi = lax.bitcast_convert_type(
        y[:, D // 2:].astype(jnp.bfloat16), jnp.uint16).astype(jnp.uint32)
    return lax.bitcast_convert_type(lo | (hi << 16), jnp.int32)


def _unpack16(xi):
    xu = lax.bitcast_convert_type(xi, jnp.uint32)
    lo = lax.bitcast_convert_type(
        (xu & 0xFFFF).astype(jnp.uint16), jnp.bfloat16)
    hi = lax.bitcast_convert_type(
        (xu >> 16).astype(jnp.uint16), jnp.bfloat16)
    return jnp.concatenate([lo, hi], axis=1)


def _ffn_body(be_ref, xs_ref, w1_ref, w3_ref, w2_ref, wrow_ref, out_ref,
              acc_ref):
    j = pl.program_id(1)
    xb = _unpack16(xs_ref[...]).astype(jnp.float32)
    a = jnp.dot(xb, w1_ref[0], preferred_element_type=jnp.float32)
    c = jnp.dot(xb, w3_ref[0], preferred_element_type=jnp.float32)
    h = a * jax.nn.sigmoid(a) * c
    part = jnp.dot(h, w2_ref[0], preferred_element_type=jnp.float32)

    @pl.when(j == 0)
    def _():
        acc_ref[...] = jnp.zeros_like(acc_ref)

    acc_ref[...] += part

    @pl.when(j == NF - 1)
    def _():
        out_ref[...] = _pack16(acc_ref[...] * wrow_ref[:, 0:1])


def _grouped_ffn(xs, W1b, W3b, W2b, w2d, block_expert):
    grid_spec = pltpu.PrefetchScalarGridSpec(
        num_scalar_prefetch=1,
        grid=(NB, NF),
        in_specs=[
            pl.BlockSpec((BM, D // 2), lambda i, j, be: (i, 0)),
            pl.BlockSpec((1, D, BF), lambda i, j, be: (be[i], 0, j)),
            pl.BlockSpec((1, D, BF), lambda i, j, be: (be[i], 0, j)),
            pl.BlockSpec((1, BF, D), lambda i, j, be: (be[i], j, 0)),
            pl.BlockSpec((BM, 128), lambda i, j, be: (i, 0)),
        ],
        out_specs=pl.BlockSpec((BM, D // 2), lambda i, j, be: (i, 0)),
        scratch_shapes=[pltpu.VMEM((BM, D), jnp.float32)],
    )
    return pl.pallas_call(
        _ffn_body,
        grid_spec=grid_spec,
        out_shape=jax.ShapeDtypeStruct((M_PAD, D // 2), jnp.int32),
        compiler_params=pltpu.CompilerParams(
            dimension_semantics=("arbitrary", "arbitrary")),
    )(block_expert, xs, W1b, W3b, W2b, w2d)


NC = 2
NS = 16
NW = NC * NS

GCH = 64
CCH = 32


def _sc_gather(table, gather_idx, n_rows, row_elems, dtype):
    rpw = n_rows // NW
    row_bytes = row_elems * 4
    max_c = min(104, (500 * 1024) // (2 * row_bytes))
    gch = 8
    for c in range(max_c - max_c % 8, 7, -8):
        if rpw % c == 0 and (rpw // c) % 2 == 0:
            gch = c
            break
    nch = rpw // gch
    mesh = plsc.VectorSubcoreMesh(core_axis_name="c", subcore_axis_name="s")

    @functools.partial(
        pl.kernel, mesh=mesh,
        out_type=jax.ShapeDtypeStruct((n_rows, row_elems), dtype),
        scratch_types=[
            pltpu.VMEM((rpw,), jnp.int32),
            pltpu.VMEM((gch, row_elems), dtype),
            pltpu.VMEM((gch, row_elems), dtype),
            pltpu.SemaphoreType.DMA,
            pltpu.SemaphoreType.DMA,
        ],
    )
    def k(x_hbm, idx_hbm, out_hbm, idx_all, ra, rb, sa, sb):
        wid = lax.axis_index("s") * NC + lax.axis_index("c")
        base = wid * rpw
        pltpu.sync_copy(idx_hbm.at[pl.ds(base, rpw)], idx_all)

        def fire(i, rows_v, sem):
            pltpu.async_copy(
                x_hbm.at[idx_all.at[pl.ds(i * gch, gch)]], rows_v, sem)

        def drain(i, rows_v, sem):
            off = base + i * gch
            pltpu.make_async_copy(x_hbm.at[pl.ds(0, gch)], rows_v, sem).wait()
            pltpu.sync_copy(rows_v, out_hbm.at[pl.ds(off, gch)])

        fire(0, ra, sa)

        def body(h, _):
            i = h * 2
            fire(i + 1, rb, sb)
            drain(i, ra, sa)

            @pl.when(i + 2 < nch)
            def _():
                fire(i + 2, ra, sa)

            drain(i + 1, rb, sb)
            return 0

        lax.fori_loop(0, nch // 2, body, 0)

    return k(table, gather_idx)


BR2 = 512


def _pair_add_body(g0_ref, g1_ref, o_ref):
    y0 = _unpack16(g0_ref[...]).astype(jnp.float32)
    y1 = _unpack16(g1_ref[...]).astype(jnp.float32)
    o_ref[...] = y0 + y1


def _pair_add(gcat):
    nb = N // BR2
    return pl.pallas_call(
        _pair_add_body,
        grid=(nb,),
        in_specs=[
            pl.BlockSpec((BR2, D // 2), lambda i: (i, 0)),
            pl.BlockSpec((BR2, D // 2), lambda i: (i + N // BR2, 0)),
        ],
        out_specs=pl.BlockSpec((BR2, D), lambda i: (i, 0)),
        out_shape=jax.ShapeDtypeStruct((N, D), jnp.float32),
    )(gcat, gcat)


def kernel(x, Wg, W1, W2, W3):
    Bb, Tt, Dd = x.shape
    x_flat = x.reshape(-1, Dd)
    top_idx, top_w, xpack = _router(x_flat, Wg)
    gather_idx, w_pad, inv_pos, block_expert = _metadata(
        top_idx[:, :K], top_w[:, :K])

    xs = _sc_gather(xpack, gather_idx, M_PAD, D // 2, jnp.int32)
    w2d = jnp.broadcast_to(w_pad[:, None], (M_PAD, 128))

    ys = _grouped_ffn(xs, W1, W3, W2, w2d, block_expert)

    gcat = _sc_gather(ys, inv_pos, N * K, D // 2, jnp.int32)
    out = _pair_add(gcat)
    return out.reshape(Bb, Tt, Dd)


# speedup vs baseline: 1.2932x; 1.0006x over previous
import functools

import jax
import jax.numpy as jnp
from jax import lax
from jax.experimental import pallas as pl
from jax.experimental.pallas import tpu as pltpu
from jax.experimental.pallas import tpu_sc as plsc

N = 8192
D = 1024
F = 4096
E = 8
K = 2

BM = 512
BF = 1024
NF = F // BF
NB = 39
M_PAD = NB * BM

BR = 1024
NEG = -1e30


def _router_body(x_ref, wg_ref, idx_ref, w_ref, xp_ref):
    xb = x_ref[...]
    lo = lax.bitcast_convert_type(
        xb[:, :D // 2].astype(jnp.bfloat16), jnp.uint16).astype(jnp.uint32)
    hi = lax.bitcast_convert_type(
        xb[:, D // 2:].astype(jnp.bfloat16), jnp.uint16).astype(jnp.uint32)
    xp_ref[...] = lax.bitcast_convert_type(lo | (hi << 16), jnp.int32)
    g = jnp.dot(xb, wg_ref[...], preferred_element_type=jnp.float32)
    lane = lax.broadcasted_iota(jnp.int32, g.shape, 1)
    valid = lane < E
    gm = jnp.where(valid, g, NEG)
    m1 = jnp.max(gm, axis=1, keepdims=True)
    i1 = jnp.min(jnp.where(gm == m1, lane, 999), axis=1, keepdims=True)
    g2 = jnp.where(lane == i1, NEG, gm)
    m2 = jnp.max(g2, axis=1, keepdims=True)
    i2 = jnp.min(jnp.where(g2 == m2, lane, 999), axis=1, keepdims=True)
    e2 = jnp.exp(m2 - m1)
    w1 = 1.0 / (1.0 + e2)
    w2 = e2 * w1
    lane8 = lax.broadcasted_iota(jnp.int32, (BR, E), 1)
    idx_ref[...] = jnp.where(lane8 == 0, i1, i2)
    w_ref[...] = jnp.where(lane8 == 0, w1, w2)


def _router(x_flat, Wg):
    wg_pad = jnp.zeros((D, 128), jnp.float32).at[:, :E].set(Wg)
    return pl.pallas_call(
        _router_body,
        grid=(N // BR,),
        in_specs=[
            pl.BlockSpec((BR, D), lambda i: (i, 0)),
            pl.BlockSpec((D, 128), lambda i: (0, 0)),
        ],
        out_specs=[
            pl.BlockSpec((BR, E), lambda i: (i, 0)),
            pl.BlockSpec((BR, E), lambda i: (i, 0)),
            pl.BlockSpec((BR, D // 2), lambda i: (i, 0)),
        ],
        out_shape=[
            jax.ShapeDtypeStruct((N, E), jnp.int32),
            jax.ShapeDtypeStruct((N, E), jnp.float32),
            jax.ShapeDtypeStruct((N, D // 2), jnp.int32),
        ],
    )(x_flat, wg_pad)


def _metadata(top_idx, top_w):
    e_flat = top_idx.T.reshape(-1)
    w_flat = top_w.T.reshape(-1)
    order = jnp.argsort(e_flat, stable=True)
    e_sorted = e_flat[order]
    offs = jnp.searchsorted(e_sorted, jnp.arange(E, dtype=e_sorted.dtype),
                            side="left").astype(jnp.int32)
    counts = jnp.diff(jnp.concatenate([offs, jnp.array([N * K], jnp.int32)]))
    nb_e = (counts + BM - 1) // BM
    blk_end = jnp.cumsum(nb_e).astype(jnp.int32)
    blk_start = blk_end - nb_e
    row_start = blk_start * BM
    onehot = (e_flat[:, None] == jnp.arange(E, dtype=e_flat.dtype)[None, :])
    csum = jnp.cumsum(onehot.astype(jnp.int32), axis=0)
    within = jnp.take_along_axis(
        csum, e_flat[:, None].astype(jnp.int32), axis=1)[:, 0] - 1
    inv_pos = row_start[e_flat] + within
    p = jnp.arange(M_PAD, dtype=jnp.int32)
    be_row = jnp.minimum(
        jnp.searchsorted(blk_end, p // BM, side="right").astype(jnp.int32),
        E - 1)
    in_grp = p - row_start[be_row]
    valid = in_grp < counts[be_row]
    rank = jnp.clip(offs[be_row] + in_grp, 0, N * K - 1)
    src = order[rank].astype(jnp.int32)
    gather_idx = jnp.where(valid, src % N, 0).astype(jnp.int32)
    w_pad = jnp.where(valid, w_flat[src], 0.0)
    block_expert = be_row[::BM]
    return gather_idx, w_pad, inv_pos, block_expert


def _pack16(y):
    lo = lax.bitcast_convert_type(
        y[:, :D // 2].astype(jnp.bfloat16), jnp.uint16).astype(jnp.uint32)
    hi = lax.bitcast_convert_type(
        y[:, D // 2:].astype(jnp.bfloat16), jnp.uint16).astype(jnp.uint32)
    return lax.bitcast_convert_type(lo | (hi << 16), jnp.int32)


def _unpack16(xi):
    xu = lax.bitcast_convert_type(xi, jnp.uint32)
    lo = lax.bitcast_convert_type(
        (xu & 0xFFFF).astype(jnp.uint16), jnp.bfloat16)
    hi = lax.bitcast_convert_type(
        (xu >> 16).astype(jnp.uint16), jnp.bfloat16)
    return jnp.concatenate([lo, hi], axis=1)


def _ffn_body(be_ref, xs_ref, w1_ref, w3_ref, w2_ref, wrow_ref, out_ref,
              acc_ref):
    j = pl.program_id(1)
    xb = _unpack16(xs_ref[...]).astype(jnp.float32)
    a = jnp.dot(xb, w1_ref[0], preferred_element_type=jnp.float32)
    c = jnp.dot(xb, w3_ref[0], preferred_element_type=jnp.float32)
    h = a * jax.nn.sigmoid(a) * c
    part = jnp.dot(h, w2_ref[0], preferred_element_type=jnp.float32)

    @pl.when(j == 0)
    def _():
        acc_ref[...] = jnp.zeros_like(acc_ref)

    acc_ref[...] += part

    @pl.when(j == NF - 1)
    def _():
        out_ref[...] = _pack16(acc_ref[...] * wrow_ref[:, 0:1])


def _grouped_ffn(xs, W1b, W3b, W2b, w2d, block_expert):
    grid_spec = pltpu.PrefetchScalarGridSpec(
        num_scalar_prefetch=1,
        grid=(NB, NF),
        in_specs=[
            pl.BlockSpec((BM, D // 2), lambda i, j, be: (i, 0)),
            pl.BlockSpec((1, D, BF), lambda i, j, be: (be[i], 0, j)),
            pl.BlockSpec((1, D, BF), lambda i, j, be: (be[i], 0, j)),
            pl.BlockSpec((1, BF, D), lambda i, j, be: (be[i], j, 0)),
            pl.BlockSpec((BM, 128), lambda i, j, be: (i, 0)),
        ],
        out_specs=pl.BlockSpec((BM, D // 2), lambda i, j, be: (i, 0)),
        scratch_shapes=[pltpu.VMEM((BM, D), jnp.float32)],
    )
    return pl.pallas_call(
        _ffn_body,
        grid_spec=grid_spec,
        out_shape=jax.ShapeDtypeStruct((M_PAD, D // 2), jnp.int32),
        compiler_params=pltpu.CompilerParams(
            dimension_semantics=("arbitrary", "arbitrary")),
    )(block_expert, xs, W1b, W3b, W2b, w2d)


NC = 2
NS = 16
NW = NC * NS


def _sc_gather(table, gather_idx, n_rows, row_elems, dtype):
    rpw = n_rows // NW
    row_bytes = row_elems * 4
    max_c = min(104, (500 * 1024) // (2 * row_bytes))
    gch = 8
    for c in range(max_c - max_c % 8, 7, -8):
        if rpw % c == 0 and (rpw // c) % 2 == 0:
            gch = c
            break
    nch = rpw // gch
    mesh = plsc.VectorSubcoreMesh(core_axis_name="c", subcore_axis_name="s")

    @functools.partial(
        pl.kernel, mesh=mesh,
        out_type=jax.ShapeDtypeStruct((n_rows, row_elems), dtype),
        scratch_types=[
            pltpu.VMEM((rpw,), jnp.int32),
            pltpu.VMEM((gch, row_elems), dtype),
            pltpu.VMEM((gch, row_elems), dtype),
            pltpu.SemaphoreType.DMA,
            pltpu.SemaphoreType.DMA,
        ],
    )
    def k(x_hbm, idx_hbm, out_hbm, idx_all, ra, rb, sa, sb):
        wid = lax.axis_index("s") * NC + lax.axis_index("c")
        base = wid * rpw
        pltpu.sync_copy(idx_hbm.at[pl.ds(base, rpw)], idx_all)

        def fire(i, rows_v, sem):
            pltpu.async_copy(
                x_hbm.at[idx_all.at[pl.ds(i * gch, gch)]], rows_v, sem)

        def drain(i, rows_v, sem):
            off = base + i * gch
            pltpu.make_async_copy(x_hbm.at[pl.ds(0, gch)], rows_v, sem).wait()
            pltpu.sync_copy(rows_v, out_hbm.at[pl.ds(off, gch)])

        fire(0, ra, sa)

        def body(h, _):
            i = h * 2
            fire(i + 1, rb, sb)
            drain(i, ra, sa)

            @pl.when(i + 2 < nch)
            def _():
                fire(i + 2, ra, sa)

            drain(i + 1, rb, sb)
            return 0

        lax.fori_loop(0, nch // 2, body, 0)

    return k(table, gather_idx)


BR2 = 512


def _pair_add_body(g0_ref, g1_ref, o_ref):
    y0 = _unpack16(g0_ref[...]).astype(jnp.float32)
    y1 = _unpack16(g1_ref[...]).astype(jnp.float32)
    o_ref[...] = y0 + y1


def _pair_add(gcat):
    nb = N // BR2
    return pl.pallas_call(
        _pair_add_body,
        grid=(nb,),
        in_specs=[
            pl.BlockSpec((BR2, D // 2), lambda i: (i, 0)),
            pl.BlockSpec((BR2, D // 2), lambda i: (i + N // BR2, 0)),
        ],
        out_specs=pl.BlockSpec((BR2, D), lambda i: (i, 0)),
        out_shape=jax.ShapeDtypeStruct((N, D), jnp.float32),
    )(gcat, gcat)


def kernel(x, Wg, W1, W2, W3):
    Bb, Tt, Dd = x.shape
    x_flat = x.reshape(-1, Dd)
    top_idx, top_w, xpack = _router(x_flat, Wg)
    gather_idx, w_pad, inv_pos, block_expert = _metadata(
        top_idx[:, :K], top_w[:, :K])

    xs = _sc_gather(xpack, gather_idx, M_PAD, D // 2, jnp.int32)
    w2d = jnp.broadcast_to(w_pad[:, None], (M_PAD, 128))

    ys = _grouped_ffn(xs, W1, W3, W2, w2d, block_expert)

    gcat = _sc_gather(ys, inv_pos, N * K, D // 2, jnp.int32)
    out = _pair_add(gcat)
    return out.reshape(Bb, Tt, Dd)
